# baseline (speedup 1.0000x reference)
import functools

import jax
import jax.numpy as jnp
from jax import lax
from jax.experimental import pallas as pl
from jax.experimental.pallas import tpu as pltpu
from jax.experimental.pallas import tpu_sc as plsc

_K = 64
_LANES = 128
_BH = 128
_L = 16


def _bf16_rne(x):
    xi = jax.lax.bitcast_convert_type(x, jnp.int32)
    r = (xi + 0x7FFF + ((xi >> 16) & 1)) & jnp.int32(-65536)
    return jax.lax.bitcast_convert_type(r, jnp.float32)


def _tc_body(cn_ref, x_ref, idx_ref):
    r0 = x_ref[0, 0]
    g0 = x_ref[0, 1]
    b0 = x_ref[0, 2]
    nrm = jnp.sqrt(r0 * r0 + g0 * g0 + b0 * b0)
    r = _bf16_rne(r0 / nrm)
    g = _bf16_rne(g0 / nrm)
    b = _bf16_rne(b0 / nrm)
    best_s = jnp.full(r.shape, -1.0, jnp.float32)
    best_i = jnp.zeros(r.shape, jnp.int32)
    for k in range(_K):
        s = r * cn_ref[k, 0] + g * cn_ref[k, 1] + b * cn_ref[k, 2]
        m = s > best_s
        best_s = jnp.where(m, s, best_s)
        best_i = jnp.where(m, jnp.int32(k), best_i)
    nz = (r0 + g0 + b0) > 0.0
    idx_ref[0] = jnp.where(nz, best_i, jnp.int32(_K))


def _sc_gather(hw, chunk, tbl_hbm, idx_hbm, out_hbm, tbl_v, idx_v, out_v):
    wid = lax.axis_index("s") * 2 + lax.axis_index("c")
    per_b = hw // chunk
    b = wid // per_b
    off = (wid % per_b) * chunk
    pltpu.sync_copy(tbl_hbm, tbl_v)
    pltpu.sync_copy(idx_hbm.at[pl.ds(wid * chunk, chunk)], idx_v)

    def body(i, _):
        iv = idx_v[pl.ds(i * _L, _L)]
        for c in range(3):
            out_v[pl.ds(c * chunk + i * _L, _L)] = plsc.load_gather(
                tbl_v, [iv + jnp.int32(72 * c)])
        return 0

    lax.fori_loop(0, chunk // _L, body, 0)
    for c in range(3):
        pltpu.sync_copy(
            out_v.at[pl.ds(c * chunk, chunk)],
            out_hbm.at[pl.ds((b * 3 + c) * hw + off, chunk)])


def kernel(rgb_mask, colors):
    B, C, H, W = rgb_mask.shape
    hw = H * W
    rows = hw // _LANES
    x = rgb_mask.reshape(B, C, rows, _LANES)
    a_norm = jnp.linalg.norm(colors, ord=2, axis=-1)
    cn = colors / a_norm[:, None]
    cnr = _bf16_rne(cn)
    idx = pl.pallas_call(
        _tc_body,
        grid=(B, rows // _BH),
        in_specs=[
            pl.BlockSpec(memory_space=pltpu.SMEM),
            pl.BlockSpec((1, C, _BH, _LANES), lambda i, j: (i, 0, j, 0)),
        ],
        out_specs=pl.BlockSpec((1, _BH, _LANES), lambda i, j: (i, j, 0)),
        out_shape=jax.ShapeDtypeStruct((B, rows, _LANES), jnp.int32),
    )(cnr, x)
    idx = idx.reshape(B * hw)

    tbl = jnp.concatenate([cn, jnp.zeros((8, 3), jnp.float32)], axis=0)
    tbl = tbl.T.reshape(3 * 72)

    chunk = (B * hw) // 32
    mesh = plsc.VectorSubcoreMesh(core_axis_name="c", subcore_axis_name="s")
    out = pl.kernel(
        functools.partial(_sc_gather, hw, chunk),
        mesh=mesh,
        out_type=jax.ShapeDtypeStruct((B * C * hw,), jnp.float32),
        scratch_types=[
            pltpu.VMEM((3 * 72,), jnp.float32),
            pltpu.VMEM((chunk,), jnp.int32),
            pltpu.VMEM((3 * chunk,), jnp.float32),
        ],
        compiler_params=pltpu.CompilerParams(needs_layout_passes=False),
    )(tbl, idx)
    return out.reshape(B, C, H, W)
